# Initial kernel scaffold; baseline (speedup 1.0000x reference)
import jax
import jax.numpy as jnp
from jax.experimental import pallas as pl


def kernel(xp, edge_index_p, ep_feats, coord_p, xl, edge_index_l, el_feats, coord_l, edge_index_c, ec_feats, coord_c, params):
    raise NotImplementedError("write your pallas kernel here")


# speedup vs baseline: 1.6879x; 1.6879x over previous
import functools

import jax
import jax.numpy as jnp
from jax import lax
from jax.experimental import pallas as pl
from jax.experimental.pallas import tpu as pltpu
from jax.experimental.pallas import tpu_sc as plsc

_D = 256
_BN = 1000
_BE = 1000
_C = 40
_NSUB = 16
_NCORE = 2
_NW = _NSUB * _NCORE
_XW = 16
_TW = 384


def _silu(x):
    return x * jax.nn.sigmoid(x)


def _dot(a, b):
    return jnp.dot(a, b, preferred_element_type=jnp.float32)


def _prep_weights(params):
    gs = [('Wp_edge', 'bp_edge', 'blk_p'), ('Wl_edge', 'bl_edge', 'blk_l'),
          ('Wc_edge', 'bc_edge', 'blk_c')]
    ins = []
    for wk, bk, blk in gs:
        ins += [params[wk], params[bk].reshape(1, _D),
                params[blk]['We1'], params[blk]['be1'],
                params[blk]['bx'].reshape(2, 1)]

    def body(*refs):
        irefs, orefs = refs[:15], refs[15:]
        for g in range(3):
            we_r, be_r, we1_r, be1_r, bx_r = irefs[5 * g:5 * g + 5]
            o_r = orefs[g]
            for i in range(2):
                wmat = we1_r[i, 2 * _D + 1:, :]
                o_r[i, 0:16, :] = _dot(we_r[...], wmat)
                o_r[i, 16:17, :] = _dot(be_r[...], wmat) + be1_r[i:i + 1, :]
                o_r[i, 17:18, :] = we1_r[i, 2 * _D:2 * _D + 1, :]
                o_r[i, 18:19, :] = jnp.broadcast_to(bx_r[i:i + 1, :], (1, _D))
                o_r[i, 19:24, :] = jnp.zeros((5, _D), jnp.float32)

    out_shape = [jax.ShapeDtypeStruct((2, 24, _D), jnp.float32)] * 3
    return pl.pallas_call(body, out_shape=out_shape)(*ins)


def _init_node(x, W, b, g, bb):
    N, F = x.shape

    def body(x_r, w_r, b_r, g_r, bb_r, o_r):
        h = _dot(x_r[...], w_r[...]) + b_r[...]
        mu = jnp.mean(h, axis=-1, keepdims=True)
        hm = h - mu
        v = jnp.mean(hm * hm, axis=-1, keepdims=True)
        o_r[...] = hm * lax.rsqrt(v + 1e-5) * g_r[...] + bb_r[...]

    return pl.pallas_call(
        body,
        grid=(N // _BN,),
        in_specs=[pl.BlockSpec((_BN, F), lambda i: (i, 0)),
                  pl.BlockSpec((F, _D), lambda i: (0, 0)),
                  pl.BlockSpec((1, _D), lambda i: (0, 0)),
                  pl.BlockSpec((1, _D), lambda i: (0, 0)),
                  pl.BlockSpec((1, _D), lambda i: (0, 0))],
        out_specs=pl.BlockSpec((_BN, _D), lambda i: (i, 0)),
        out_shape=jax.ShapeDtypeStruct((N, _D), jnp.float32),
    )(x, W, b.reshape(1, _D), g.reshape(1, _D), bb.reshape(1, _D))


def _ab_prep(h, xpad, whd, whs):
    N = h.shape[0]

    def body(h_r, x_r, a_w, b_w, a_o, b_o):
        hv = h_r[...]
        xv = x_r[...]
        z = jnp.zeros((_BN, _TW - _D - _XW), jnp.float32)
        a_o[...] = jnp.concatenate([_dot(hv, a_w[...]), xv, z], axis=-1)
        b_o[...] = jnp.concatenate([_dot(hv, b_w[...]), xv, z], axis=-1)

    return pl.pallas_call(
        body,
        grid=(N // _BN,),
        in_specs=[pl.BlockSpec((_BN, _D), lambda i: (i, 0)),
                  pl.BlockSpec((_BN, _XW), lambda i: (i, 0)),
                  pl.BlockSpec((_D, _D), lambda i: (0, 0)),
                  pl.BlockSpec((_D, _D), lambda i: (0, 0))],
        out_specs=[pl.BlockSpec((_BN, _TW), lambda i: (i, 0))] * 2,
        out_shape=[jax.ShapeDtypeStruct((N, _TW), jnp.float32)] * 2,
    )(h, xpad, whd, whs)


def _edge_mlp(ga, gb, feats, wext, we2, be2, wx_row, with_coef):
    E = ga.shape[0]

    def body(ga_r, gb_r, ft_r, wx_r, be2_r, wext_r, we2_r, *outs):
        gav = ga_r[...]
        gbv = gb_r[...]
        rel = gav[:, _D:_D + _XW] - gbv[:, _D:_D + _XW]
        d2 = jnp.sum(rel * rel, axis=-1, keepdims=True)
        wc = wext_r[0:16, :]
        bc = wext_r[16:17, :]
        wd2 = wext_r[17:18, :]
        pre = (gav[:, :_D] + gbv[:, :_D] + _dot(ft_r[...], wc) + bc +
               d2 * wd2)
        m1 = _silu(pre)
        m = _silu(_dot(m1, we2_r[...]) + be2_r[...])
        outs[0][...] = m[:, :128]
        outs[1][...] = m[:, 128:]
        if with_coef:
            bx = wext_r[18:19, 0:1]
            coef = jnp.sum(m * wx_r[...], axis=-1, keepdims=True) + bx
            outs[2][...] = jnp.concatenate(
                [rel * coef, jnp.zeros((_BE, 128 - _XW), jnp.float32)],
                axis=-1)

    out_shape = [jax.ShapeDtypeStruct((E, 128), jnp.float32),
                 jax.ShapeDtypeStruct((E, 128), jnp.float32)]
    out_specs = [pl.BlockSpec((_BE, 128), lambda i: (i, 0)),
                 pl.BlockSpec((_BE, 128), lambda i: (i, 0))]
    if with_coef:
        out_shape.append(jax.ShapeDtypeStruct((E, 128), jnp.float32))
        out_specs.append(pl.BlockSpec((_BE, 128), lambda i: (i, 0)))

    return pl.pallas_call(
        body,
        grid=(E // _BE,),
        in_specs=[pl.BlockSpec((_BE, _TW), lambda i: (i, 0)),
                  pl.BlockSpec((_BE, _TW), lambda i: (i, 0)),
                  pl.BlockSpec((_BE, 16), lambda i: (i, 0)),
                  pl.BlockSpec((1, _D), lambda i: (0, 0)),
                  pl.BlockSpec((1, _D), lambda i: (0, 0)),
                  pl.BlockSpec((24, _D), lambda i: (0, 0)),
                  pl.BlockSpec((_D, _D), lambda i: (0, 0))],
        out_specs=out_specs,
        out_shape=out_shape,
    )(ga, gb, feats, wx_row, be2.reshape(1, _D), wext, we2)


def _node_update(h, aga, agb, w1a, w1ba, w1bb, b1, w2, b2, x, dx0, dx1,
                 with_x):
    N = h.shape[0]

    def body(*refs):
        if with_x:
            (h_r, aa_r, ab_r, w1a_r, w1ba_r, w1bb_r, b1_r, w2_r, b2_r,
             x_r, dx0_r, dx1_r, ho_r, xo_r) = refs
        else:
            (h_r, aa_r, ab_r, w1a_r, w1ba_r, w1bb_r, b1_r, w2_r, b2_r,
             ho_r) = refs
        hv = h_r[...]
        t = (_dot(hv, w1a_r[...]) + _dot(aa_r[...], w1ba_r[...]) +
             _dot(ab_r[...], w1bb_r[...]) + b1_r[...])
        t = _silu(t)
        ho_r[...] = hv + _dot(t, w2_r[...]) + b2_r[...]
        if with_x:
            dx = (dx0_r[...] + dx1_r[...])[:, :_XW]
            xo_r[...] = x_r[...] + dx * (1.0 / 16.0)

    in_specs = [pl.BlockSpec((_BN, _D), lambda i: (i, 0)),
                pl.BlockSpec((_BN, 128), lambda i: (i, 0)),
                pl.BlockSpec((_BN, 128), lambda i: (i, 0)),
                pl.BlockSpec((_D, _D), lambda i: (0, 0)),
                pl.BlockSpec((128, _D), lambda i: (0, 0)),
                pl.BlockSpec((128, _D), lambda i: (0, 0)),
                pl.BlockSpec((1, _D), lambda i: (0, 0)),
                pl.BlockSpec((_D, _D), lambda i: (0, 0)),
                pl.BlockSpec((1, _D), lambda i: (0, 0))]
    out_specs = [pl.BlockSpec((_BN, _D), lambda i: (i, 0))]
    out_shape = [jax.ShapeDtypeStruct((N, _D), jnp.float32)]
    args = [h, aga, agb, w1a, w1ba, w1bb, b1.reshape(1, _D), w2,
            b2.reshape(1, _D)]
    if with_x:
        in_specs += [pl.BlockSpec((_BN, _XW), lambda i: (i, 0)),
                     pl.BlockSpec((_BN, 128), lambda i: (i, 0)),
                     pl.BlockSpec((_BN, 128), lambda i: (i, 0))]
        out_specs.append(pl.BlockSpec((_BN, _XW), lambda i: (i, 0)))
        out_shape.append(jax.ShapeDtypeStruct((N, _XW), jnp.float32))
        args += [x, dx0, dx1]

    res = pl.pallas_call(
        body, grid=(N // _BN,), in_specs=in_specs, out_specs=out_specs,
        out_shape=out_shape)(*args)
    return res if with_x else (res[0], None)


def _sc_gather(td, ts, src, dst):
    E = src.shape[0]
    epw = E // _NW
    nch = epw // _C
    mesh = plsc.VectorSubcoreMesh(core_axis_name="c", subcore_axis_name="s")

    @functools.partial(
        pl.kernel, mesh=mesh,
        out_type=[jax.ShapeDtypeStruct((E, _TW), jnp.float32),
                  jax.ShapeDtypeStruct((E, _TW), jnp.float32)],
        scratch_types=[pltpu.VMEM((_C,), jnp.int32),
                       pltpu.VMEM((_C,), jnp.int32),
                       pltpu.VMEM((_C, _TW), jnp.float32),
                       pltpu.VMEM((_C, _TW), jnp.float32),
                       pltpu.SemaphoreType.DMA, pltpu.SemaphoreType.DMA])
    def k(a_hbm, b_hbm, src_hbm, dst_hbm, ga_hbm, gb_hbm,
          sidx, didx, abuf, bbuf, s1, s2):
        wid = lax.axis_index("s") * _NCORE + lax.axis_index("c")
        base = wid * epw

        @pl.loop(0, nch)
        def _(i):
            eb = base + i * _C
            pltpu.sync_copy(src_hbm.at[pl.ds(eb, _C)], sidx)
            pltpu.sync_copy(dst_hbm.at[pl.ds(eb, _C)], didx)
            c1 = pltpu.async_copy(a_hbm.at[didx], abuf, s1)
            c2 = pltpu.async_copy(b_hbm.at[sidx], bbuf, s2)
            c1.wait()
            c2.wait()
            pltpu.sync_copy(abuf, ga_hbm.at[pl.ds(eb, _C)])
            pltpu.sync_copy(bbuf, gb_hbm.at[pl.ds(eb, _C)])

    return k(td, ts, src, dst)


def _sc_scatter_m(ma, mb, dst, N):
    E = dst.shape[0]
    ept = E // _NSUB
    nch = ept // _C
    rpt = (N // _NSUB) // 8 * 8
    tail = N - _NSUB * rpt
    mesh = plsc.VectorSubcoreMesh(core_axis_name="c", subcore_axis_name="s")
    z128 = jnp.zeros((N, 128), jnp.float32)

    @functools.partial(
        pl.kernel, mesh=mesh,
        out_type=[jax.ShapeDtypeStruct((N, 128), jnp.float32),
                  jax.ShapeDtypeStruct((N, 128), jnp.float32)],
        scratch_types=[pltpu.VMEM((_C,), jnp.int32),
                       pltpu.VMEM((_C, 128), jnp.float32),
                       pltpu.VMEM_SHARED((N, 128), jnp.float32)])
    def k(ma_hbm, mb_hbm, dst_hbm, z_hbm, aa_hbm, ab_hbm, didx, rows, acc):
        cid = lax.axis_index("c")
        sid = lax.axis_index("s")

        @pl.when(sid == 0)
        def _():
            pltpu.sync_copy(z_hbm, acc)

        plsc.subcore_barrier()
        base = sid * ept

        @pl.loop(0, nch)
        def _(i):
            eb = base + i * _C
            pltpu.sync_copy(dst_hbm.at[pl.ds(eb, _C)], didx)

            @pl.when(cid == 0)
            def _():
                pltpu.sync_copy(ma_hbm.at[pl.ds(eb, _C)], rows)

            @pl.when(cid == 1)
            def _():
                pltpu.sync_copy(mb_hbm.at[pl.ds(eb, _C)], rows)

            pltpu.sync_copy(rows, acc.at[didx], add=True)

        plsc.subcore_barrier()
        rb = sid * rpt

        def wout(o_hbm):
            pltpu.sync_copy(acc.at[pl.ds(rb, rpt)], o_hbm.at[pl.ds(rb, rpt)])
            if tail:
                @pl.when(sid == 0)
                def _():
                    tb = _NSUB * rpt
                    pltpu.sync_copy(acc.at[pl.ds(tb, tail)],
                                    o_hbm.at[pl.ds(tb, tail)])

        @pl.when(cid == 0)
        def _():
            wout(aa_hbm)

        @pl.when(cid == 1)
        def _():
            wout(ab_hbm)

    return k(ma, mb, dst, z128)


def _sc_scatter_rc(rc, dst, N):
    E = dst.shape[0]
    eph = E // 2
    ept = eph // _NSUB
    nch = ept // _C
    rpt = (N // _NSUB) // 8 * 8
    tail = N - _NSUB * rpt
    mesh = plsc.VectorSubcoreMesh(core_axis_name="c", subcore_axis_name="s")
    z128 = jnp.zeros((N, 128), jnp.float32)

    @functools.partial(
        pl.kernel, mesh=mesh,
        out_type=[jax.ShapeDtypeStruct((N, 128), jnp.float32),
                  jax.ShapeDtypeStruct((N, 128), jnp.float32)],
        scratch_types=[pltpu.VMEM((_C,), jnp.int32),
                       pltpu.VMEM((_C, 128), jnp.float32),
                       pltpu.VMEM_SHARED((N, 128), jnp.float32)])
    def k(rc_hbm, dst_hbm, z_hbm, d0_hbm, d1_hbm, didx, rows, acc):
        cid = lax.axis_index("c")
        sid = lax.axis_index("s")

        @pl.when(sid == 0)
        def _():
            pltpu.sync_copy(z_hbm, acc)

        plsc.subcore_barrier()
        base = cid * eph + sid * ept

        @pl.loop(0, nch)
        def _(i):
            eb = base + i * _C
            pltpu.sync_copy(dst_hbm.at[pl.ds(eb, _C)], didx)
            pltpu.sync_copy(rc_hbm.at[pl.ds(eb, _C)], rows)
            pltpu.sync_copy(rows, acc.at[didx], add=True)

        plsc.subcore_barrier()
        rb = sid * rpt

        def wout(o_hbm):
            pltpu.sync_copy(acc.at[pl.ds(rb, rpt)], o_hbm.at[pl.ds(rb, rpt)])
            if tail:
                @pl.when(sid == 0)
                def _():
                    tb = _NSUB * rpt
                    pltpu.sync_copy(acc.at[pl.ds(tb, tail)],
                                    o_hbm.at[pl.ds(tb, tail)])

        @pl.when(cid == 0)
        def _():
            wout(d0_hbm)

        @pl.when(cid == 1)
        def _():
            wout(d1_hbm)

    return k(rc, dst, z128)


def _egnn_block(h, xpad, feats, src, dst, blk, wext, i, with_x):
    N = h.shape[0]
    whd = blk['We1'][i, :_D, :]
    whs = blk['We1'][i, _D:2 * _D, :]
    wx_row = blk['Wx'][i].reshape(1, _D)
    w1a = blk['Wh1'][i, :_D, :]
    w1ba = blk['Wh1'][i, _D:_D + 128, :]
    w1bb = blk['Wh1'][i, _D + 128:, :]

    td, ts = _ab_prep(h, xpad, whd, whs)
    ga, gb = _sc_gather(td, ts, src, dst)
    res = _edge_mlp(ga, gb, feats, wext[i], blk['We2'][i],
                    blk['be2'][i], wx_row, with_coef=with_x)
    ma, mb = res[0], res[1]
    aga, agb = _sc_scatter_m(ma, mb, dst, N)
    if with_x:
        dx0, dx1 = _sc_scatter_rc(res[2], dst, N)
    else:
        dx0 = dx1 = None
    h_new, x_new = _node_update(h, aga, agb, w1a, w1ba, w1bb, blk['bh1'][i],
                                blk['Wh2'][i], blk['bh2'][i], xpad, dx0, dx1,
                                with_x)
    return h_new, x_new


def kernel(xp, edge_index_p, ep_feats, coord_p, xl, edge_index_l, el_feats,
           coord_l, edge_index_c, ec_feats, coord_c, params):
    NP = xp.shape[0]
    L = params['blk_p']['We1'].shape[0]

    def pad_x(c):
        n = c.shape[0]
        return jnp.concatenate(
            [c, jnp.zeros((n, _XW - c.shape[1]), jnp.float32)], axis=1)

    sp, dp = (edge_index_p[0].astype(jnp.int32),
              edge_index_p[1].astype(jnp.int32))
    sl, dl = (edge_index_l[0].astype(jnp.int32),
              edge_index_l[1].astype(jnp.int32))
    sc, dc = (edge_index_c[0].astype(jnp.int32),
              edge_index_c[1].astype(jnp.int32))

    wext_p, wext_l, wext_c = _prep_weights(params)
    hp = _init_node(xp, params['Wp_node'], params['bp_node'],
                    params['ln_p_g'], params['ln_p_b'])
    hl = _init_node(xl, params['Wl_node'], params['bl_node'],
                    params['ln_l_g'], params['ln_l_b'])
    xpp, xpl, xpc = pad_x(coord_p), pad_x(coord_l), pad_x(coord_c)

    for i in range(L):
        last = i == L - 1
        hp, xpp = _egnn_block(hp, xpp, ep_feats, sp, dp, params['blk_p'],
                              wext_p, i, with_x=not last)
        hl, xpl = _egnn_block(hl, xpl, el_feats, sl, dl, params['blk_l'],
                              wext_l, i, with_x=not last)
        hc = jnp.concatenate([hp, hl], axis=0)
        hc, xpc = _egnn_block(hc, xpc, ec_feats, sc, dc, params['blk_c'],
                              wext_c, i, with_x=not last)
        hp = hc[:NP]
        hl = hc[NP:]

    return hp, hl, hc


# speedup vs baseline: 1.8645x; 1.1046x over previous
import functools

import jax
import jax.numpy as jnp
from jax import lax
from jax.experimental import pallas as pl
from jax.experimental.pallas import tpu as pltpu
from jax.experimental.pallas import tpu_sc as plsc

_D = 256
_BN = 1000
_BE = 1000
_C = 40
_NSUB = 16
_NCORE = 2
_NW = _NSUB * _NCORE
_XW = 16
_TW = 256


def _pack2(lo, hi):
    lo_u = lax.bitcast_convert_type(lo.astype(jnp.bfloat16),
                                    jnp.uint16).astype(jnp.uint32)
    hi_u = lax.bitcast_convert_type(hi.astype(jnp.bfloat16),
                                    jnp.uint16).astype(jnp.uint32)
    return lax.bitcast_convert_type(lo_u | (hi_u << 16), jnp.float32)


def _unpack2(w):
    wu = lax.bitcast_convert_type(w, jnp.uint32)
    lo = lax.bitcast_convert_type((wu & 0xFFFF).astype(jnp.uint16),
                                  jnp.bfloat16).astype(jnp.float32)
    hi = lax.bitcast_convert_type((wu >> 16).astype(jnp.uint16),
                                  jnp.bfloat16).astype(jnp.float32)
    return lo, hi


def _silu(x):
    return x * jax.nn.sigmoid(x)


def _dot(a, b):
    return jnp.dot(a, b, preferred_element_type=jnp.float32)


def _prep_weights(params):
    gs = [('Wp_edge', 'bp_edge', 'blk_p'), ('Wl_edge', 'bl_edge', 'blk_l'),
          ('Wc_edge', 'bc_edge', 'blk_c')]
    ins = []
    for wk, bk, blk in gs:
        ins += [params[wk], params[bk].reshape(1, _D),
                params[blk]['We1'], params[blk]['be1'],
                params[blk]['bx'].reshape(2, 1)]

    def body(*refs):
        irefs, orefs = refs[:15], refs[15:]
        for g in range(3):
            we_r, be_r, we1_r, be1_r, bx_r = irefs[5 * g:5 * g + 5]
            o_r = orefs[g]
            for i in range(2):
                wmat = we1_r[i, 2 * _D + 1:, :]
                o_r[i, 0:16, :] = _dot(we_r[...], wmat)
                o_r[i, 16:17, :] = _dot(be_r[...], wmat) + be1_r[i:i + 1, :]
                o_r[i, 17:18, :] = we1_r[i, 2 * _D:2 * _D + 1, :]
                o_r[i, 18:19, :] = jnp.broadcast_to(bx_r[i:i + 1, :], (1, _D))
                o_r[i, 19:24, :] = jnp.zeros((5, _D), jnp.float32)

    out_shape = [jax.ShapeDtypeStruct((2, 24, _D), jnp.float32)] * 3
    return pl.pallas_call(body, out_shape=out_shape)(*ins)


def _init_node(x, W, b, g, bb):
    N, F = x.shape

    def body(x_r, w_r, b_r, g_r, bb_r, o_r):
        h = _dot(x_r[...], w_r[...]) + b_r[...]
        mu = jnp.mean(h, axis=-1, keepdims=True)
        hm = h - mu
        v = jnp.mean(hm * hm, axis=-1, keepdims=True)
        o_r[...] = hm * lax.rsqrt(v + 1e-5) * g_r[...] + bb_r[...]

    return pl.pallas_call(
        body,
        grid=(N // _BN,),
        in_specs=[pl.BlockSpec((_BN, F), lambda i: (i, 0)),
                  pl.BlockSpec((F, _D), lambda i: (0, 0)),
                  pl.BlockSpec((1, _D), lambda i: (0, 0)),
                  pl.BlockSpec((1, _D), lambda i: (0, 0)),
                  pl.BlockSpec((1, _D), lambda i: (0, 0))],
        out_specs=pl.BlockSpec((_BN, _D), lambda i: (i, 0)),
        out_shape=jax.ShapeDtypeStruct((N, _D), jnp.float32),
    )(x, W, b.reshape(1, _D), g.reshape(1, _D), bb.reshape(1, _D))


def _ab_prep(h, xpad, whd, whs):
    N = h.shape[0]

    def body(h_r, x_r, a_w, b_w, a_o, b_o):
        hv = h_r[...]
        xv = x_r[...]
        zx = jnp.zeros((_BN, 128 - _XW), jnp.float32)
        for o_r, w_r in ((a_o, a_w), (b_o, b_w)):
            av = _dot(hv, w_r[...])
            o_r[...] = jnp.concatenate(
                [_pack2(av[:, :128], av[:, 128:]), xv, zx], axis=-1)

    return pl.pallas_call(
        body,
        grid=(N // _BN,),
        in_specs=[pl.BlockSpec((_BN, _D), lambda i: (i, 0)),
                  pl.BlockSpec((_BN, _XW), lambda i: (i, 0)),
                  pl.BlockSpec((_D, _D), lambda i: (0, 0)),
                  pl.BlockSpec((_D, _D), lambda i: (0, 0))],
        out_specs=[pl.BlockSpec((_BN, _TW), lambda i: (i, 0))] * 2,
        out_shape=[jax.ShapeDtypeStruct((N, _TW), jnp.float32)] * 2,
    )(h, xpad, whd, whs)


def _edge_mlp(ga, gb, feats, wext, we2, be2, wx_row, with_coef):
    E = ga.shape[0]

    def body(ga_r, gb_r, ft_r, wx_r, be2_r, wext_r, we2_r, *outs):
        gav = ga_r[...]
        gbv = gb_r[...]
        ga0, ga1 = _unpack2(gav[:, :128])
        gb0, gb1 = _unpack2(gbv[:, :128])
        xd = gav[:, 128:128 + _XW]
        xs = gbv[:, 128:128 + _XW]
        rel = xd - xs
        d2 = jnp.sum(rel * rel, axis=-1, keepdims=True)
        wc = wext_r[0:16, :]
        bc = wext_r[16:17, :]
        wd2 = wext_r[17:18, :]
        gsum = jnp.concatenate([ga0 + gb0, ga1 + gb1], axis=-1)
        pre = gsum + _dot(ft_r[...], wc) + bc + d2 * wd2
        m1 = _silu(pre)
        m = _silu(_dot(m1, we2_r[...]) + be2_r[...])
        outs[0][...] = m[:, :128]
        outs[1][...] = m[:, 128:]
        if with_coef:
            bx = wext_r[18:19, 0:1]
            coef = jnp.sum(m * wx_r[...], axis=-1, keepdims=True) + bx
            outs[2][...] = jnp.concatenate(
                [rel * coef, jnp.zeros((_BE, 128 - _XW), jnp.float32)],
                axis=-1)

    out_shape = [jax.ShapeDtypeStruct((E, 128), jnp.float32),
                 jax.ShapeDtypeStruct((E, 128), jnp.float32)]
    out_specs = [pl.BlockSpec((_BE, 128), lambda i: (i, 0)),
                 pl.BlockSpec((_BE, 128), lambda i: (i, 0))]
    if with_coef:
        out_shape.append(jax.ShapeDtypeStruct((E, 128), jnp.float32))
        out_specs.append(pl.BlockSpec((_BE, 128), lambda i: (i, 0)))

    return pl.pallas_call(
        body,
        grid=(E // _BE,),
        in_specs=[pl.BlockSpec((_BE, _TW), lambda i: (i, 0)),
                  pl.BlockSpec((_BE, _TW), lambda i: (i, 0)),
                  pl.BlockSpec((_BE, 16), lambda i: (i, 0)),
                  pl.BlockSpec((1, _D), lambda i: (0, 0)),
                  pl.BlockSpec((1, _D), lambda i: (0, 0)),
                  pl.BlockSpec((24, _D), lambda i: (0, 0)),
                  pl.BlockSpec((_D, _D), lambda i: (0, 0))],
        out_specs=out_specs,
        out_shape=out_shape,
    )(ga, gb, feats, wx_row, be2.reshape(1, _D), wext, we2)


def _node_update(h, aga, agb, w1a, w1ba, w1bb, b1, w2, b2, x, dx0, dx1,
                 with_x):
    N = h.shape[0]

    def body(*refs):
        if with_x:
            (h_r, aa_r, ab_r, w1a_r, w1ba_r, w1bb_r, b1_r, w2_r, b2_r,
             x_r, dx0_r, dx1_r, ho_r, xo_r) = refs
        else:
            (h_r, aa_r, ab_r, w1a_r, w1ba_r, w1bb_r, b1_r, w2_r, b2_r,
             ho_r) = refs
        hv = h_r[...]
        t = (_dot(hv, w1a_r[...]) + _dot(aa_r[...], w1ba_r[...]) +
             _dot(ab_r[...], w1bb_r[...]) + b1_r[...])
        t = _silu(t)
        ho_r[...] = hv + _dot(t, w2_r[...]) + b2_r[...]
        if with_x:
            dx = (dx0_r[...] + dx1_r[...])[:, :_XW]
            xo_r[...] = x_r[...] + dx * (1.0 / 16.0)

    in_specs = [pl.BlockSpec((_BN, _D), lambda i: (i, 0)),
                pl.BlockSpec((_BN, 128), lambda i: (i, 0)),
                pl.BlockSpec((_BN, 128), lambda i: (i, 0)),
                pl.BlockSpec((_D, _D), lambda i: (0, 0)),
                pl.BlockSpec((128, _D), lambda i: (0, 0)),
                pl.BlockSpec((128, _D), lambda i: (0, 0)),
                pl.BlockSpec((1, _D), lambda i: (0, 0)),
                pl.BlockSpec((_D, _D), lambda i: (0, 0)),
                pl.BlockSpec((1, _D), lambda i: (0, 0))]
    out_specs = [pl.BlockSpec((_BN, _D), lambda i: (i, 0))]
    out_shape = [jax.ShapeDtypeStruct((N, _D), jnp.float32)]
    args = [h, aga, agb, w1a, w1ba, w1bb, b1.reshape(1, _D), w2,
            b2.reshape(1, _D)]
    if with_x:
        in_specs += [pl.BlockSpec((_BN, _XW), lambda i: (i, 0)),
                     pl.BlockSpec((_BN, 128), lambda i: (i, 0)),
                     pl.BlockSpec((_BN, 128), lambda i: (i, 0))]
        out_specs.append(pl.BlockSpec((_BN, _XW), lambda i: (i, 0)))
        out_shape.append(jax.ShapeDtypeStruct((N, _XW), jnp.float32))
        args += [x, dx0, dx1]

    res = pl.pallas_call(
        body, grid=(N // _BN,), in_specs=in_specs, out_specs=out_specs,
        out_shape=out_shape)(*args)
    return res if with_x else (res[0], None)


def _sc_gather(td, ts, src, dst):
    E = src.shape[0]
    epw = E // _NW
    nch = epw // _C
    mesh = plsc.VectorSubcoreMesh(core_axis_name="c", subcore_axis_name="s")

    @functools.partial(
        pl.kernel, mesh=mesh,
        out_type=[jax.ShapeDtypeStruct((E, _TW), jnp.float32),
                  jax.ShapeDtypeStruct((E, _TW), jnp.float32)],
        scratch_types=[pltpu.VMEM((_C,), jnp.int32),
                       pltpu.VMEM((_C,), jnp.int32),
                       pltpu.VMEM((_C, _TW), jnp.float32),
                       pltpu.VMEM((_C, _TW), jnp.float32),
                       pltpu.SemaphoreType.DMA, pltpu.SemaphoreType.DMA])
    def k(a_hbm, b_hbm, src_hbm, dst_hbm, ga_hbm, gb_hbm,
          sidx, didx, abuf, bbuf, s1, s2):
        wid = lax.axis_index("s") * _NCORE + lax.axis_index("c")
        base = wid * epw

        @pl.loop(0, nch)
        def _(i):
            eb = base + i * _C
            pltpu.sync_copy(src_hbm.at[pl.ds(eb, _C)], sidx)
            pltpu.sync_copy(dst_hbm.at[pl.ds(eb, _C)], didx)
            c1 = pltpu.async_copy(a_hbm.at[didx], abuf, s1)
            c2 = pltpu.async_copy(b_hbm.at[sidx], bbuf, s2)
            c1.wait()
            c2.wait()
            pltpu.sync_copy(abuf, ga_hbm.at[pl.ds(eb, _C)])
            pltpu.sync_copy(bbuf, gb_hbm.at[pl.ds(eb, _C)])

    return k(td, ts, src, dst)


def _sc_scatter_m(ma, mb, dst, N):
    E = dst.shape[0]
    ept = E // _NSUB
    nch = ept // _C
    rpt = (N // _NSUB) // 8 * 8
    tail = N - _NSUB * rpt
    mesh = plsc.VectorSubcoreMesh(core_axis_name="c", subcore_axis_name="s")
    z128 = jnp.zeros((N, 128), jnp.float32)

    @functools.partial(
        pl.kernel, mesh=mesh,
        out_type=[jax.ShapeDtypeStruct((N, 128), jnp.float32),
                  jax.ShapeDtypeStruct((N, 128), jnp.float32)],
        scratch_types=[pltpu.VMEM((_C,), jnp.int32),
                       pltpu.VMEM((_C, 128), jnp.float32),
                       pltpu.VMEM_SHARED((N, 128), jnp.float32)])
    def k(ma_hbm, mb_hbm, dst_hbm, z_hbm, aa_hbm, ab_hbm, didx, rows, acc):
        cid = lax.axis_index("c")
        sid = lax.axis_index("s")

        @pl.when(sid == 0)
        def _():
            pltpu.sync_copy(z_hbm, acc)

        plsc.subcore_barrier()
        base = sid * ept

        @pl.loop(0, nch)
        def _(i):
            eb = base + i * _C
            pltpu.sync_copy(dst_hbm.at[pl.ds(eb, _C)], didx)

            @pl.when(cid == 0)
            def _():
                pltpu.sync_copy(ma_hbm.at[pl.ds(eb, _C)], rows)

            @pl.when(cid == 1)
            def _():
                pltpu.sync_copy(mb_hbm.at[pl.ds(eb, _C)], rows)

            pltpu.sync_copy(rows, acc.at[didx], add=True)

        plsc.subcore_barrier()
        rb = sid * rpt

        def wout(o_hbm):
            pltpu.sync_copy(acc.at[pl.ds(rb, rpt)], o_hbm.at[pl.ds(rb, rpt)])
            if tail:
                @pl.when(sid == 0)
                def _():
                    tb = _NSUB * rpt
                    pltpu.sync_copy(acc.at[pl.ds(tb, tail)],
                                    o_hbm.at[pl.ds(tb, tail)])

        @pl.when(cid == 0)
        def _():
            wout(aa_hbm)

        @pl.when(cid == 1)
        def _():
            wout(ab_hbm)

    return k(ma, mb, dst, z128)


def _sc_scatter_rc(rc, dst, N):
    E = dst.shape[0]
    eph = E // 2
    ept = eph // _NSUB
    nch = ept // _C
    rpt = (N // _NSUB) // 8 * 8
    tail = N - _NSUB * rpt
    mesh = plsc.VectorSubcoreMesh(core_axis_name="c", subcore_axis_name="s")
    z128 = jnp.zeros((N, 128), jnp.float32)

    @functools.partial(
        pl.kernel, mesh=mesh,
        out_type=[jax.ShapeDtypeStruct((N, 128), jnp.float32),
                  jax.ShapeDtypeStruct((N, 128), jnp.float32)],
        scratch_types=[pltpu.VMEM((_C,), jnp.int32),
                       pltpu.VMEM((_C, 128), jnp.float32),
                       pltpu.VMEM_SHARED((N, 128), jnp.float32)])
    def k(rc_hbm, dst_hbm, z_hbm, d0_hbm, d1_hbm, didx, rows, acc):
        cid = lax.axis_index("c")
        sid = lax.axis_index("s")

        @pl.when(sid == 0)
        def _():
            pltpu.sync_copy(z_hbm, acc)

        plsc.subcore_barrier()
        base = cid * eph + sid * ept

        @pl.loop(0, nch)
        def _(i):
            eb = base + i * _C
            pltpu.sync_copy(dst_hbm.at[pl.ds(eb, _C)], didx)
            pltpu.sync_copy(rc_hbm.at[pl.ds(eb, _C)], rows)
            pltpu.sync_copy(rows, acc.at[didx], add=True)

        plsc.subcore_barrier()
        rb = sid * rpt

        def wout(o_hbm):
            pltpu.sync_copy(acc.at[pl.ds(rb, rpt)], o_hbm.at[pl.ds(rb, rpt)])
            if tail:
                @pl.when(sid == 0)
                def _():
                    tb = _NSUB * rpt
                    pltpu.sync_copy(acc.at[pl.ds(tb, tail)],
                                    o_hbm.at[pl.ds(tb, tail)])

        @pl.when(cid == 0)
        def _():
            wout(d0_hbm)

        @pl.when(cid == 1)
        def _():
            wout(d1_hbm)

    return k(rc, dst, z128)


def _egnn_block(h, xpad, feats, src, dst, blk, wext, i, with_x):
    N = h.shape[0]
    whd = blk['We1'][i, :_D, :]
    whs = blk['We1'][i, _D:2 * _D, :]
    wx_row = blk['Wx'][i].reshape(1, _D)
    w1a = blk['Wh1'][i, :_D, :]
    w1ba = blk['Wh1'][i, _D:_D + 128, :]
    w1bb = blk['Wh1'][i, _D + 128:, :]

    td, ts = _ab_prep(h, xpad, whd, whs)
    ga, gb = _sc_gather(td, ts, src, dst)
    res = _edge_mlp(ga, gb, feats, wext[i], blk['We2'][i],
                    blk['be2'][i], wx_row, with_coef=with_x)
    ma, mb = res[0], res[1]
    aga, agb = _sc_scatter_m(ma, mb, dst, N)
    if with_x:
        dx0, dx1 = _sc_scatter_rc(res[2], dst, N)
    else:
        dx0 = dx1 = None
    h_new, x_new = _node_update(h, aga, agb, w1a, w1ba, w1bb, blk['bh1'][i],
                                blk['Wh2'][i], blk['bh2'][i], xpad, dx0, dx1,
                                with_x)
    return h_new, x_new


def kernel(xp, edge_index_p, ep_feats, coord_p, xl, edge_index_l, el_feats,
           coord_l, edge_index_c, ec_feats, coord_c, params):
    NP = xp.shape[0]
    L = params['blk_p']['We1'].shape[0]

    def pad_x(c):
        n = c.shape[0]
        return jnp.concatenate(
            [c, jnp.zeros((n, _XW - c.shape[1]), jnp.float32)], axis=1)

    sp, dp = (edge_index_p[0].astype(jnp.int32),
              edge_index_p[1].astype(jnp.int32))
    sl, dl = (edge_index_l[0].astype(jnp.int32),
              edge_index_l[1].astype(jnp.int32))
    sc, dc = (edge_index_c[0].astype(jnp.int32),
              edge_index_c[1].astype(jnp.int32))

    wext_p, wext_l, wext_c = _prep_weights(params)
    hp = _init_node(xp, params['Wp_node'], params['bp_node'],
                    params['ln_p_g'], params['ln_p_b'])
    hl = _init_node(xl, params['Wl_node'], params['bl_node'],
                    params['ln_l_g'], params['ln_l_b'])
    xpp, xpl, xpc = pad_x(coord_p), pad_x(coord_l), pad_x(coord_c)

    for i in range(L):
        last = i == L - 1
        hp, xpp = _egnn_block(hp, xpp, ep_feats, sp, dp, params['blk_p'],
                              wext_p, i, with_x=not last)
        hl, xpl = _egnn_block(hl, xpl, el_feats, sl, dl, params['blk_l'],
                              wext_l, i, with_x=not last)
        hc = jnp.concatenate([hp, hl], axis=0)
        hc, xpc = _egnn_block(hc, xpc, ec_feats, sc, dc, params['blk_c'],
                              wext_c, i, with_x=not last)
        hp = hc[:NP]
        hl = hc[NP:]

    return hp, hl, hc


# speedup vs baseline: 2.0827x; 1.1171x over previous
import functools

import jax
import jax.numpy as jnp
from jax import lax
from jax.experimental import pallas as pl
from jax.experimental.pallas import tpu as pltpu
from jax.experimental.pallas import tpu_sc as plsc

_D = 256
_BN = 1000
_BE = 1000
_C = 40
_NSUB = 16
_NCORE = 2
_NW = _NSUB * _NCORE
_XW = 16
_TW = 256


def _pack2(lo, hi):
    lo_u = lax.bitcast_convert_type(lo.astype(jnp.bfloat16),
                                    jnp.uint16).astype(jnp.uint32)
    hi_u = lax.bitcast_convert_type(hi.astype(jnp.bfloat16),
                                    jnp.uint16).astype(jnp.uint32)
    return lax.bitcast_convert_type(lo_u | (hi_u << 16), jnp.float32)


def _unpack2(w):
    wu = lax.bitcast_convert_type(w, jnp.uint32)
    lo = lax.bitcast_convert_type((wu & 0xFFFF).astype(jnp.uint16),
                                  jnp.bfloat16).astype(jnp.float32)
    hi = lax.bitcast_convert_type((wu >> 16).astype(jnp.uint16),
                                  jnp.bfloat16).astype(jnp.float32)
    return lo, hi


def _silu(x):
    return x * jax.nn.sigmoid(x)


def _dot(a, b):
    return jnp.dot(a, b, preferred_element_type=jnp.float32)


def _prep_weights(params):
    gs = [('Wp_edge', 'bp_edge', 'blk_p'), ('Wl_edge', 'bl_edge', 'blk_l'),
          ('Wc_edge', 'bc_edge', 'blk_c')]
    ins = []
    for wk, bk, blk in gs:
        ins += [params[wk], params[bk].reshape(1, _D),
                params[blk]['We1'], params[blk]['be1'],
                params[blk]['bx'].reshape(2, 1)]

    def body(*refs):
        irefs, orefs = refs[:15], refs[15:]
        for g in range(3):
            we_r, be_r, we1_r, be1_r, bx_r = irefs[5 * g:5 * g + 5]
            o_r = orefs[g]
            for i in range(2):
                wmat = we1_r[i, 2 * _D + 1:, :]
                o_r[i, 0:16, :] = _dot(we_r[...], wmat)
                o_r[i, 16:17, :] = _dot(be_r[...], wmat) + be1_r[i:i + 1, :]
                o_r[i, 17:18, :] = we1_r[i, 2 * _D:2 * _D + 1, :]
                o_r[i, 18:19, :] = jnp.broadcast_to(bx_r[i:i + 1, :], (1, _D))
                o_r[i, 19:24, :] = jnp.zeros((5, _D), jnp.float32)

    out_shape = [jax.ShapeDtypeStruct((2, 24, _D), jnp.float32)] * 3
    return pl.pallas_call(body, out_shape=out_shape)(*ins)


def _init_node(x, W, b, g, bb):
    N, F = x.shape

    def body(x_r, w_r, b_r, g_r, bb_r, o_r):
        h = _dot(x_r[...], w_r[...]) + b_r[...]
        mu = jnp.mean(h, axis=-1, keepdims=True)
        hm = h - mu
        v = jnp.mean(hm * hm, axis=-1, keepdims=True)
        o_r[...] = hm * lax.rsqrt(v + 1e-5) * g_r[...] + bb_r[...]

    return pl.pallas_call(
        body,
        grid=(N // _BN,),
        in_specs=[pl.BlockSpec((_BN, F), lambda i: (i, 0)),
                  pl.BlockSpec((F, _D), lambda i: (0, 0)),
                  pl.BlockSpec((1, _D), lambda i: (0, 0)),
                  pl.BlockSpec((1, _D), lambda i: (0, 0)),
                  pl.BlockSpec((1, _D), lambda i: (0, 0))],
        out_specs=pl.BlockSpec((_BN, _D), lambda i: (i, 0)),
        out_shape=jax.ShapeDtypeStruct((N, _D), jnp.float32),
    )(x, W, b.reshape(1, _D), g.reshape(1, _D), bb.reshape(1, _D))


def _ab_prep(h, xpad, whd, whs):
    N = h.shape[0]

    def body(h_r, x_r, a_w, b_w, a_o, b_o):
        hv = h_r[...]
        xv = x_r[...]
        zx = jnp.zeros((_BN, 128 - _XW), jnp.float32)
        for o_r, w_r in ((a_o, a_w), (b_o, b_w)):
            av = _dot(hv, w_r[...])
            o_r[...] = jnp.concatenate(
                [_pack2(av[:, :128], av[:, 128:]), xv, zx], axis=-1)

    return pl.pallas_call(
        body,
        grid=(N // _BN,),
        in_specs=[pl.BlockSpec((_BN, _D), lambda i: (i, 0)),
                  pl.BlockSpec((_BN, _XW), lambda i: (i, 0)),
                  pl.BlockSpec((_D, _D), lambda i: (0, 0)),
                  pl.BlockSpec((_D, _D), lambda i: (0, 0))],
        out_specs=[pl.BlockSpec((_BN, _TW), lambda i: (i, 0))] * 2,
        out_shape=[jax.ShapeDtypeStruct((N, _TW), jnp.float32)] * 2,
    )(h, xpad, whd, whs)


def _edge_mlp(ga, gb, xd, xs, feats, wext, we2, be2, wx_row, with_coef):
    E = ga.shape[0]

    def body(ga_r, gb_r, xd_r, xs_r, ft_r, wx_r, be2_r, wext_r, we2_r,
             *outs):
        ga0, ga1 = _unpack2(ga_r[...])
        gb0, gb1 = _unpack2(gb_r[...])
        rel = xd_r[...] - xs_r[...]
        d2 = jnp.sum(rel * rel, axis=-1, keepdims=True)
        wc = wext_r[0:16, :]
        bc = wext_r[16:17, :]
        wd2 = wext_r[17:18, :]
        gsum = jnp.concatenate([ga0 + gb0, ga1 + gb1], axis=-1)
        pre = gsum + _dot(ft_r[...], wc) + bc + d2 * wd2
        m1 = _silu(pre)
        m = _silu(_dot(m1, we2_r[...]) + be2_r[...])
        outs[0][...] = m[:, :128]
        outs[1][...] = m[:, 128:]
        if with_coef:
            bx = wext_r[18:19, 0:1]
            coef = jnp.sum(m * wx_r[...], axis=-1, keepdims=True) + bx
            outs[2][...] = jnp.concatenate(
                [rel * coef, jnp.zeros((_BE, 128 - _XW), jnp.float32)],
                axis=-1)

    out_shape = [jax.ShapeDtypeStruct((E, 128), jnp.float32),
                 jax.ShapeDtypeStruct((E, 128), jnp.float32)]
    out_specs = [pl.BlockSpec((_BE, 128), lambda i: (i, 0)),
                 pl.BlockSpec((_BE, 128), lambda i: (i, 0))]
    if with_coef:
        out_shape.append(jax.ShapeDtypeStruct((E, 128), jnp.float32))
        out_specs.append(pl.BlockSpec((_BE, 128), lambda i: (i, 0)))

    return pl.pallas_call(
        body,
        grid=(E // _BE,),
        in_specs=[pl.BlockSpec((_BE, 128), lambda i: (i, 0)),
                  pl.BlockSpec((_BE, 128), lambda i: (i, 0)),
                  pl.BlockSpec((_BE, _XW), lambda i: (i, 0)),
                  pl.BlockSpec((_BE, _XW), lambda i: (i, 0)),
                  pl.BlockSpec((_BE, 16), lambda i: (i, 0)),
                  pl.BlockSpec((1, _D), lambda i: (0, 0)),
                  pl.BlockSpec((1, _D), lambda i: (0, 0)),
                  pl.BlockSpec((24, _D), lambda i: (0, 0)),
                  pl.BlockSpec((_D, _D), lambda i: (0, 0))],
        out_specs=out_specs,
        out_shape=out_shape,
    )(ga, gb, xd, xs, feats, wx_row, be2.reshape(1, _D), wext, we2)


def _node_update(h, aga, agb, w1a, w1ba, w1bb, b1, w2, b2, x, dx0, dx1,
                 with_x):
    N = h.shape[0]

    def body(*refs):
        if with_x:
            (h_r, aa_r, ab_r, w1a_r, w1ba_r, w1bb_r, b1_r, w2_r, b2_r,
             x_r, dx0_r, dx1_r, ho_r, xo_r) = refs
        else:
            (h_r, aa_r, ab_r, w1a_r, w1ba_r, w1bb_r, b1_r, w2_r, b2_r,
             ho_r) = refs
        hv = h_r[...]
        t = (_dot(hv, w1a_r[...]) + _dot(aa_r[...], w1ba_r[...]) +
             _dot(ab_r[...], w1bb_r[...]) + b1_r[...])
        t = _silu(t)
        ho_r[...] = hv + _dot(t, w2_r[...]) + b2_r[...]
        if with_x:
            dx = (dx0_r[...] + dx1_r[...])[:, :_XW]
            xo_r[...] = x_r[...] + dx * (1.0 / 16.0)

    in_specs = [pl.BlockSpec((_BN, _D), lambda i: (i, 0)),
                pl.BlockSpec((_BN, 128), lambda i: (i, 0)),
                pl.BlockSpec((_BN, 128), lambda i: (i, 0)),
                pl.BlockSpec((_D, _D), lambda i: (0, 0)),
                pl.BlockSpec((128, _D), lambda i: (0, 0)),
                pl.BlockSpec((128, _D), lambda i: (0, 0)),
                pl.BlockSpec((1, _D), lambda i: (0, 0)),
                pl.BlockSpec((_D, _D), lambda i: (0, 0)),
                pl.BlockSpec((1, _D), lambda i: (0, 0))]
    out_specs = [pl.BlockSpec((_BN, _D), lambda i: (i, 0))]
    out_shape = [jax.ShapeDtypeStruct((N, _D), jnp.float32)]
    args = [h, aga, agb, w1a, w1ba, w1bb, b1.reshape(1, _D), w2,
            b2.reshape(1, _D)]
    if with_x:
        in_specs += [pl.BlockSpec((_BN, _XW), lambda i: (i, 0)),
                     pl.BlockSpec((_BN, 128), lambda i: (i, 0)),
                     pl.BlockSpec((_BN, 128), lambda i: (i, 0))]
        out_specs.append(pl.BlockSpec((_BN, _XW), lambda i: (i, 0)))
        out_shape.append(jax.ShapeDtypeStruct((N, _XW), jnp.float32))
        args += [x, dx0, dx1]

    res = pl.pallas_call(
        body, grid=(N // _BN,), in_specs=in_specs, out_specs=out_specs,
        out_shape=out_shape)(*args)
    return res if with_x else (res[0], None)


def _sc_gather(td, ts, src, dst):
    E = src.shape[0]
    ept = E // _NW
    CG = 64
    nfull = ept // CG
    ctail = ept - nfull * CG
    npairs = nfull // 2
    rem = nfull % 2
    mesh = plsc.VectorSubcoreMesh(core_axis_name="c", subcore_axis_name="s")

    @functools.partial(
        pl.kernel, mesh=mesh,
        out_type=[jax.ShapeDtypeStruct((E, 128), jnp.float32),
                  jax.ShapeDtypeStruct((E, 128), jnp.float32),
                  jax.ShapeDtypeStruct((E, _XW), jnp.float32),
                  jax.ShapeDtypeStruct((E, _XW), jnp.float32)],
        scratch_types=[pltpu.VMEM((CG,), jnp.int32),
                       pltpu.VMEM((CG,), jnp.int32),
                       pltpu.VMEM((CG,), jnp.int32),
                       pltpu.VMEM((CG,), jnp.int32),
                       pltpu.VMEM((CG, _TW), jnp.float32),
                       pltpu.VMEM((CG, _TW), jnp.float32),
                       pltpu.VMEM((CG, _TW), jnp.float32),
                       pltpu.VMEM((CG, _TW), jnp.float32),
                       pltpu.VMEM((CG, _XW), jnp.float32),
                       pltpu.VMEM((CG, _XW), jnp.float32),
                       pltpu.VMEM((CG, _XW), jnp.float32),
                       pltpu.VMEM((CG, _XW), jnp.float32),
                       pltpu.SemaphoreType.DMA, pltpu.SemaphoreType.DMA,
                       pltpu.SemaphoreType.DMA, pltpu.SemaphoreType.DMA,
                       pltpu.SemaphoreType.DMA, pltpu.SemaphoreType.DMA])
    def k(a_hbm, b_hbm, src_hbm, dst_hbm, ga_hbm, gb_hbm, xd_hbm, xs_hbm,
          si0, di0, si1, di1, ab0, bb0, ab1, bb1,
          xd0, xs0, xd1, xs1,
          sa0, sb0, sa1, sb1, sw0, sw1):
        wid = lax.axis_index("s") * _NCORE + lax.axis_index("c")
        base = wid * ept

        def fetch(eb, n, si, di, ab, bb, sa, sb):
            sin = si.at[pl.ds(0, n)] if n != CG else si
            din = di.at[pl.ds(0, n)] if n != CG else di
            abn = ab.at[pl.ds(0, n)] if n != CG else ab
            bbn = bb.at[pl.ds(0, n)] if n != CG else bb
            pltpu.sync_copy(src_hbm.at[pl.ds(eb, n)], sin)
            pltpu.sync_copy(dst_hbm.at[pl.ds(eb, n)], din)
            ca = pltpu.async_copy(a_hbm.at[din], abn, sa)
            cb = pltpu.async_copy(b_hbm.at[sin], bbn, sb)
            return ca, cb

        def wback(eb, n, ab, bb, xdb, xsb, sw):
            @pl.loop(0, n)
            def _(r):
                xdb[r, pl.ds(0, _XW)] = ab[r, pl.ds(128, _XW)]
                xsb[r, pl.ds(0, _XW)] = bb[r, pl.ds(128, _XW)]

            es = pl.ds(eb, n)
            rs = pl.ds(0, n)
            xdn = xdb.at[rs] if n != CG else xdb
            xsn = xsb.at[rs] if n != CG else xsb
            return [
                pltpu.async_copy(ab.at[rs, pl.ds(0, 128)],
                                 ga_hbm.at[es], sw),
                pltpu.async_copy(bb.at[rs, pl.ds(0, 128)],
                                 gb_hbm.at[es], sw),
                pltpu.async_copy(xdn, xd_hbm.at[es], sw),
                pltpu.async_copy(xsn, xs_hbm.at[es], sw),
            ]

        @pl.loop(0, npairs)
        def _(j):
            e0 = base + j * (2 * CG)
            e1 = e0 + CG
            ca0, cb0 = fetch(e0, CG, si0, di0, ab0, bb0, sa0, sb0)
            ca1, cb1 = fetch(e1, CG, si1, di1, ab1, bb1, sa1, sb1)
            ca0.wait()
            cb0.wait()
            w0 = wback(e0, CG, ab0, bb0, xd0, xs0, sw0)
            ca1.wait()
            cb1.wait()
            w1 = wback(e1, CG, ab1, bb1, xd1, xs1, sw1)
            for w in w0 + w1:
                w.wait()

        tb = base + npairs * 2 * CG
        if rem:
            ca, cb = fetch(tb, CG, si0, di0, ab0, bb0, sa0, sb0)
            ca.wait()
            cb.wait()
            for w in wback(tb, CG, ab0, bb0, xd0, xs0, sw0):
                w.wait()
            tb = tb + CG
        if ctail:
            ca, cb = fetch(tb, ctail, si1, di1, ab1, bb1, sa1, sb1)
            ca.wait()
            cb.wait()
            for w in wback(tb, ctail, ab1, bb1, xd1, xs1, sw1):
                w.wait()

    return k(td, ts, src, dst)


def _sc_scatter_m(ma, mb, dst, N):
    E = dst.shape[0]
    ept = E // _NSUB
    nch = ept // _C
    rpt = (N // _NSUB) // 8 * 8
    tail = N - _NSUB * rpt
    mesh = plsc.VectorSubcoreMesh(core_axis_name="c", subcore_axis_name="s")
    z128 = jnp.zeros((N, 128), jnp.float32)

    @functools.partial(
        pl.kernel, mesh=mesh,
        out_type=[jax.ShapeDtypeStruct((N, 128), jnp.float32),
                  jax.ShapeDtypeStruct((N, 128), jnp.float32)],
        scratch_types=[pltpu.VMEM((_C,), jnp.int32),
                       pltpu.VMEM((_C, 128), jnp.float32),
                       pltpu.VMEM_SHARED((N, 128), jnp.float32)])
    def k(ma_hbm, mb_hbm, dst_hbm, z_hbm, aa_hbm, ab_hbm, didx, rows, acc):
        cid = lax.axis_index("c")
        sid = lax.axis_index("s")

        @pl.when(sid == 0)
        def _():
            pltpu.sync_copy(z_hbm, acc)

        plsc.subcore_barrier()
        base = sid * ept

        @pl.loop(0, nch)
        def _(i):
            eb = base + i * _C
            pltpu.sync_copy(dst_hbm.at[pl.ds(eb, _C)], didx)

            @pl.when(cid == 0)
            def _():
                pltpu.sync_copy(ma_hbm.at[pl.ds(eb, _C)], rows)

            @pl.when(cid == 1)
            def _():
                pltpu.sync_copy(mb_hbm.at[pl.ds(eb, _C)], rows)

            pltpu.sync_copy(rows, acc.at[didx], add=True)

        plsc.subcore_barrier()
        rb = sid * rpt

        def wout(o_hbm):
            pltpu.sync_copy(acc.at[pl.ds(rb, rpt)], o_hbm.at[pl.ds(rb, rpt)])
            if tail:
                @pl.when(sid == 0)
                def _():
                    tb = _NSUB * rpt
                    pltpu.sync_copy(acc.at[pl.ds(tb, tail)],
                                    o_hbm.at[pl.ds(tb, tail)])

        @pl.when(cid == 0)
        def _():
            wout(aa_hbm)

        @pl.when(cid == 1)
        def _():
            wout(ab_hbm)

    return k(ma, mb, dst, z128)


def _sc_scatter_rc(rc, dst, N):
    E = dst.shape[0]
    eph = E // 2
    ept = eph // _NSUB
    nch = ept // _C
    rpt = (N // _NSUB) // 8 * 8
    tail = N - _NSUB * rpt
    mesh = plsc.VectorSubcoreMesh(core_axis_name="c", subcore_axis_name="s")
    z128 = jnp.zeros((N, 128), jnp.float32)

    @functools.partial(
        pl.kernel, mesh=mesh,
        out_type=[jax.ShapeDtypeStruct((N, 128), jnp.float32),
                  jax.ShapeDtypeStruct((N, 128), jnp.float32)],
        scratch_types=[pltpu.VMEM((_C,), jnp.int32),
                       pltpu.VMEM((_C, 128), jnp.float32),
                       pltpu.VMEM_SHARED((N, 128), jnp.float32)])
    def k(rc_hbm, dst_hbm, z_hbm, d0_hbm, d1_hbm, didx, rows, acc):
        cid = lax.axis_index("c")
        sid = lax.axis_index("s")

        @pl.when(sid == 0)
        def _():
            pltpu.sync_copy(z_hbm, acc)

        plsc.subcore_barrier()
        base = cid * eph + sid * ept

        @pl.loop(0, nch)
        def _(i):
            eb = base + i * _C
            pltpu.sync_copy(dst_hbm.at[pl.ds(eb, _C)], didx)
            pltpu.sync_copy(rc_hbm.at[pl.ds(eb, _C)], rows)
            pltpu.sync_copy(rows, acc.at[didx], add=True)

        plsc.subcore_barrier()
        rb = sid * rpt

        def wout(o_hbm):
            pltpu.sync_copy(acc.at[pl.ds(rb, rpt)], o_hbm.at[pl.ds(rb, rpt)])
            if tail:
                @pl.when(sid == 0)
                def _():
                    tb = _NSUB * rpt
                    pltpu.sync_copy(acc.at[pl.ds(tb, tail)],
                                    o_hbm.at[pl.ds(tb, tail)])

        @pl.when(cid == 0)
        def _():
            wout(d0_hbm)

        @pl.when(cid == 1)
        def _():
            wout(d1_hbm)

    return k(rc, dst, z128)


def _egnn_block(h, xpad, feats, src, dst, blk, wext, i, with_x):
    N = h.shape[0]
    whd = blk['We1'][i, :_D, :]
    whs = blk['We1'][i, _D:2 * _D, :]
    wx_row = blk['Wx'][i].reshape(1, _D)
    w1a = blk['Wh1'][i, :_D, :]
    w1ba = blk['Wh1'][i, _D:_D + 128, :]
    w1bb = blk['Wh1'][i, _D + 128:, :]

    td, ts = _ab_prep(h, xpad, whd, whs)
    ga, gb, xd, xs = _sc_gather(td, ts, src, dst)
    res = _edge_mlp(ga, gb, xd, xs, feats, wext[i], blk['We2'][i],
                    blk['be2'][i], wx_row, with_coef=with_x)
    ma, mb = res[0], res[1]
    aga, agb = _sc_scatter_m(ma, mb, dst, N)
    if with_x:
        dx0, dx1 = _sc_scatter_rc(res[2], dst, N)
    else:
        dx0 = dx1 = None
    h_new, x_new = _node_update(h, aga, agb, w1a, w1ba, w1bb, blk['bh1'][i],
                                blk['Wh2'][i], blk['bh2'][i], xpad, dx0, dx1,
                                with_x)
    return h_new, x_new


def kernel(xp, edge_index_p, ep_feats, coord_p, xl, edge_index_l, el_feats,
           coord_l, edge_index_c, ec_feats, coord_c, params):
    NP = xp.shape[0]
    L = params['blk_p']['We1'].shape[0]

    def pad_x(c):
        n = c.shape[0]
        return jnp.concatenate(
            [c, jnp.zeros((n, _XW - c.shape[1]), jnp.float32)], axis=1)

    sp, dp = (edge_index_p[0].astype(jnp.int32),
              edge_index_p[1].astype(jnp.int32))
    sl, dl = (edge_index_l[0].astype(jnp.int32),
              edge_index_l[1].astype(jnp.int32))
    sc, dc = (edge_index_c[0].astype(jnp.int32),
              edge_index_c[1].astype(jnp.int32))

    wext_p, wext_l, wext_c = _prep_weights(params)
    hp = _init_node(xp, params['Wp_node'], params['bp_node'],
                    params['ln_p_g'], params['ln_p_b'])
    hl = _init_node(xl, params['Wl_node'], params['bl_node'],
                    params['ln_l_g'], params['ln_l_b'])
    xpp, xpl, xpc = pad_x(coord_p), pad_x(coord_l), pad_x(coord_c)

    for i in range(L):
        last = i == L - 1
        hp, xpp = _egnn_block(hp, xpp, ep_feats, sp, dp, params['blk_p'],
                              wext_p, i, with_x=not last)
        hl, xpl = _egnn_block(hl, xpl, el_feats, sl, dl, params['blk_l'],
                              wext_l, i, with_x=not last)
        hc = jnp.concatenate([hp, hl], axis=0)
        hc, xpc = _egnn_block(hc, xpc, ec_feats, sc, dc, params['blk_c'],
                              wext_c, i, with_x=not last)
        hp = hc[:NP]
        hl = hc[NP:]

    return hp, hl, hc


# speedup vs baseline: 2.6689x; 1.2814x over previous
import functools

import jax
import jax.numpy as jnp
from jax import lax
from jax.experimental import pallas as pl
from jax.experimental.pallas import tpu as pltpu
from jax.experimental.pallas import tpu_sc as plsc

_D = 256
_BN = 1000
_BE = 1000
_C = 40
_NSUB = 16
_NCORE = 2
_NW = _NSUB * _NCORE
_XW = 16
_TW = 256


def _pack2(lo, hi):
    lo_u = lax.bitcast_convert_type(lo.astype(jnp.bfloat16),
                                    jnp.uint16).astype(jnp.uint32)
    hi_u = lax.bitcast_convert_type(hi.astype(jnp.bfloat16),
                                    jnp.uint16).astype(jnp.uint32)
    return lax.bitcast_convert_type(lo_u | (hi_u << 16), jnp.float32)


def _unpack2(w):
    wu = lax.bitcast_convert_type(w, jnp.uint32)
    lo = lax.bitcast_convert_type((wu & 0xFFFF).astype(jnp.uint16),
                                  jnp.bfloat16).astype(jnp.float32)
    hi = lax.bitcast_convert_type((wu >> 16).astype(jnp.uint16),
                                  jnp.bfloat16).astype(jnp.float32)
    return lo, hi


def _silu(x):
    return x * jax.nn.sigmoid(x)


def _dot(a, b):
    return jnp.dot(a, b, preferred_element_type=jnp.float32)


def _prep_weights(params):
    gs = [('Wp_edge', 'bp_edge', 'blk_p'), ('Wl_edge', 'bl_edge', 'blk_l'),
          ('Wc_edge', 'bc_edge', 'blk_c')]
    ins = []
    for wk, bk, blk in gs:
        ins += [params[wk], params[bk].reshape(1, _D),
                params[blk]['We1'], params[blk]['be1'],
                params[blk]['bx'].reshape(2, 1)]

    def body(*refs):
        irefs, orefs = refs[:15], refs[15:]
        for g in range(3):
            we_r, be_r, we1_r, be1_r, bx_r = irefs[5 * g:5 * g + 5]
            o_r = orefs[g]
            for i in range(2):
                wmat = we1_r[i, 2 * _D + 1:, :]
                o_r[i, 0:16, :] = _dot(we_r[...], wmat)
                o_r[i, 16:17, :] = _dot(be_r[...], wmat) + be1_r[i:i + 1, :]
                o_r[i, 17:18, :] = we1_r[i, 2 * _D:2 * _D + 1, :]
                o_r[i, 18:19, :] = jnp.broadcast_to(bx_r[i:i + 1, :], (1, _D))
                o_r[i, 19:24, :] = jnp.zeros((5, _D), jnp.float32)

    out_shape = [jax.ShapeDtypeStruct((2, 24, _D), jnp.float32)] * 3
    return pl.pallas_call(body, out_shape=out_shape)(*ins)


def _init_node(x, W, b, g, bb):
    N, F = x.shape

    def body(x_r, w_r, b_r, g_r, bb_r, o_r):
        h = _dot(x_r[...], w_r[...]) + b_r[...]
        mu = jnp.mean(h, axis=-1, keepdims=True)
        hm = h - mu
        v = jnp.mean(hm * hm, axis=-1, keepdims=True)
        o_r[...] = hm * lax.rsqrt(v + 1e-5) * g_r[...] + bb_r[...]

    return pl.pallas_call(
        body,
        grid=(N // _BN,),
        in_specs=[pl.BlockSpec((_BN, F), lambda i: (i, 0)),
                  pl.BlockSpec((F, _D), lambda i: (0, 0)),
                  pl.BlockSpec((1, _D), lambda i: (0, 0)),
                  pl.BlockSpec((1, _D), lambda i: (0, 0)),
                  pl.BlockSpec((1, _D), lambda i: (0, 0))],
        out_specs=pl.BlockSpec((_BN, _D), lambda i: (i, 0)),
        out_shape=jax.ShapeDtypeStruct((N, _D), jnp.float32),
    )(x, W, b.reshape(1, _D), g.reshape(1, _D), bb.reshape(1, _D))


def _ab_prep(h, xpad, whd, whs):
    N = h.shape[0]

    def body(h_r, x_r, a_w, b_w, a_o, b_o):
        hv = h_r[...]
        xv = x_r[...]
        zx = jnp.zeros((_BN, 128 - _XW), jnp.float32)
        for o_r, w_r in ((a_o, a_w), (b_o, b_w)):
            av = _dot(hv, w_r[...])
            o_r[...] = jnp.concatenate(
                [_pack2(av[:, :128], av[:, 128:]), xv, zx], axis=-1)

    return pl.pallas_call(
        body,
        grid=(N // _BN,),
        in_specs=[pl.BlockSpec((_BN, _D), lambda i: (i, 0)),
                  pl.BlockSpec((_BN, _XW), lambda i: (i, 0)),
                  pl.BlockSpec((_D, _D), lambda i: (0, 0)),
                  pl.BlockSpec((_D, _D), lambda i: (0, 0))],
        out_specs=[pl.BlockSpec((_BN, _TW), lambda i: (i, 0))] * 2,
        out_shape=[jax.ShapeDtypeStruct((N, _TW), jnp.float32)] * 2,
    )(h, xpad, whd, whs)


def _edge_mlp(ga, gb, xd, xs, feats, wext, we2, be2, wx_row, with_coef):
    E = ga.shape[0]

    def body(ga_r, gb_r, xd_r, xs_r, ft_r, wx_r, be2_r, wext_r, we2_r,
             *outs):
        ga0, ga1 = _unpack2(ga_r[...])
        gb0, gb1 = _unpack2(gb_r[...])
        rel = xd_r[...] - xs_r[...]
        d2 = jnp.sum(rel * rel, axis=-1, keepdims=True)
        wc = wext_r[0:16, :]
        bc = wext_r[16:17, :]
        wd2 = wext_r[17:18, :]
        gsum = jnp.concatenate([ga0 + gb0, ga1 + gb1], axis=-1)
        pre = gsum + _dot(ft_r[...], wc) + bc + d2 * wd2
        m1 = _silu(pre)
        m = _silu(_dot(m1, we2_r[...]) + be2_r[...])
        outs[0][0, :, :] = m[:, :128]
        outs[0][1, :, :] = m[:, 128:]
        if with_coef:
            bx = wext_r[18:19, 0:1]
            coef = jnp.sum(m * wx_r[...], axis=-1, keepdims=True) + bx
            outs[1][...] = jnp.concatenate(
                [rel * coef, jnp.zeros((_BE, 128 - _XW), jnp.float32)],
                axis=-1)

    out_shape = [jax.ShapeDtypeStruct((2, E, 128), jnp.float32)]
    out_specs = [pl.BlockSpec((2, _BE, 128), lambda i: (0, i, 0))]
    if with_coef:
        out_shape.append(jax.ShapeDtypeStruct((E, 128), jnp.float32))
        out_specs.append(pl.BlockSpec((_BE, 128), lambda i: (i, 0)))

    return pl.pallas_call(
        body,
        grid=(E // _BE,),
        in_specs=[pl.BlockSpec((_BE, 128), lambda i: (i, 0)),
                  pl.BlockSpec((_BE, 128), lambda i: (i, 0)),
                  pl.BlockSpec((_BE, _XW), lambda i: (i, 0)),
                  pl.BlockSpec((_BE, _XW), lambda i: (i, 0)),
                  pl.BlockSpec((_BE, 16), lambda i: (i, 0)),
                  pl.BlockSpec((1, _D), lambda i: (0, 0)),
                  pl.BlockSpec((1, _D), lambda i: (0, 0)),
                  pl.BlockSpec((24, _D), lambda i: (0, 0)),
                  pl.BlockSpec((_D, _D), lambda i: (0, 0))],
        out_specs=out_specs,
        out_shape=out_shape,
    )(ga, gb, xd, xs, feats, wx_row, be2.reshape(1, _D), wext, we2)


def _node_update(h, ag2, w1a, w1ba, w1bb, b1, w2, b2, x, dx2, with_x):
    N = h.shape[0]

    def body(*refs):
        if with_x:
            (h_r, ag_r, w1a_r, w1ba_r, w1bb_r, b1_r, w2_r, b2_r,
             x_r, dx_r, ho_r, xo_r) = refs
        else:
            (h_r, ag_r, w1a_r, w1ba_r, w1bb_r, b1_r, w2_r, b2_r,
             ho_r) = refs
        hv = h_r[...]
        t = (_dot(hv, w1a_r[...]) + _dot(ag_r[0, :, :], w1ba_r[...]) +
             _dot(ag_r[1, :, :], w1bb_r[...]) + b1_r[...])
        t = _silu(t)
        ho_r[...] = hv + _dot(t, w2_r[...]) + b2_r[...]
        if with_x:
            dx = (dx_r[0, :, :] + dx_r[1, :, :])[:, :_XW]
            xo_r[...] = x_r[...] + dx * (1.0 / 16.0)

    in_specs = [pl.BlockSpec((_BN, _D), lambda i: (i, 0)),
                pl.BlockSpec((2, _BN, 128), lambda i: (0, i, 0)),
                pl.BlockSpec((_D, _D), lambda i: (0, 0)),
                pl.BlockSpec((128, _D), lambda i: (0, 0)),
                pl.BlockSpec((128, _D), lambda i: (0, 0)),
                pl.BlockSpec((1, _D), lambda i: (0, 0)),
                pl.BlockSpec((_D, _D), lambda i: (0, 0)),
                pl.BlockSpec((1, _D), lambda i: (0, 0))]
    out_specs = [pl.BlockSpec((_BN, _D), lambda i: (i, 0))]
    out_shape = [jax.ShapeDtypeStruct((N, _D), jnp.float32)]
    args = [h, ag2, w1a, w1ba, w1bb, b1.reshape(1, _D), w2,
            b2.reshape(1, _D)]
    if with_x:
        in_specs += [pl.BlockSpec((_BN, _XW), lambda i: (i, 0)),
                     pl.BlockSpec((2, _BN, 128), lambda i: (0, i, 0))]
        out_specs.append(pl.BlockSpec((_BN, _XW), lambda i: (i, 0)))
        out_shape.append(jax.ShapeDtypeStruct((N, _XW), jnp.float32))
        args += [x, dx2]

    res = pl.pallas_call(
        body, grid=(N // _BN,), in_specs=in_specs, out_specs=out_specs,
        out_shape=out_shape)(*args)
    return res if with_x else (res[0], None)


def _sc_gather(td, ts, src, dst):
    E = src.shape[0]
    ept = E // _NW
    CG = 64
    nfull = ept // CG
    ctail = ept - nfull * CG
    npairs = nfull // 2
    rem = nfull % 2
    mesh = plsc.VectorSubcoreMesh(core_axis_name="c", subcore_axis_name="s")

    @functools.partial(
        pl.kernel, mesh=mesh,
        out_type=[jax.ShapeDtypeStruct((E, 128), jnp.float32),
                  jax.ShapeDtypeStruct((E, 128), jnp.float32),
                  jax.ShapeDtypeStruct((E, _XW), jnp.float32),
                  jax.ShapeDtypeStruct((E, _XW), jnp.float32)],
        scratch_types=[pltpu.VMEM((CG,), jnp.int32),
                       pltpu.VMEM((CG,), jnp.int32),
                       pltpu.VMEM((CG,), jnp.int32),
                       pltpu.VMEM((CG,), jnp.int32),
                       pltpu.VMEM((CG, _TW), jnp.float32),
                       pltpu.VMEM((CG, _TW), jnp.float32),
                       pltpu.VMEM((CG, _TW), jnp.float32),
                       pltpu.VMEM((CG, _TW), jnp.float32),
                       pltpu.VMEM((CG, _XW), jnp.float32),
                       pltpu.VMEM((CG, _XW), jnp.float32),
                       pltpu.VMEM((CG, _XW), jnp.float32),
                       pltpu.VMEM((CG, _XW), jnp.float32),
                       pltpu.SemaphoreType.DMA, pltpu.SemaphoreType.DMA,
                       pltpu.SemaphoreType.DMA, pltpu.SemaphoreType.DMA,
                       pltpu.SemaphoreType.DMA, pltpu.SemaphoreType.DMA])
    def k(a_hbm, b_hbm, src_hbm, dst_hbm, ga_hbm, gb_hbm, xd_hbm, xs_hbm,
          si0, di0, si1, di1, ab0, bb0, ab1, bb1,
          xd0, xs0, xd1, xs1,
          sa0, sb0, sa1, sb1, sw0, sw1):
        wid = lax.axis_index("s") * _NCORE + lax.axis_index("c")
        base = wid * ept

        def fetch(eb, n, si, di, ab, bb, sa, sb):
            sin = si.at[pl.ds(0, n)] if n != CG else si
            din = di.at[pl.ds(0, n)] if n != CG else di
            abn = ab.at[pl.ds(0, n)] if n != CG else ab
            bbn = bb.at[pl.ds(0, n)] if n != CG else bb
            pltpu.sync_copy(src_hbm.at[pl.ds(eb, n)], sin)
            pltpu.sync_copy(dst_hbm.at[pl.ds(eb, n)], din)
            ca = pltpu.async_copy(a_hbm.at[din], abn, sa)
            cb = pltpu.async_copy(b_hbm.at[sin], bbn, sb)
            return ca, cb

        def wback(eb, n, ab, bb, xdb, xsb, sw):
            @pl.loop(0, n)
            def _(r):
                xdb[r, pl.ds(0, _XW)] = ab[r, pl.ds(128, _XW)]
                xsb[r, pl.ds(0, _XW)] = bb[r, pl.ds(128, _XW)]

            es = pl.ds(eb, n)
            rs = pl.ds(0, n)
            xdn = xdb.at[rs] if n != CG else xdb
            xsn = xsb.at[rs] if n != CG else xsb
            return [
                pltpu.async_copy(ab.at[rs, pl.ds(0, 128)],
                                 ga_hbm.at[es], sw),
                pltpu.async_copy(bb.at[rs, pl.ds(0, 128)],
                                 gb_hbm.at[es], sw),
                pltpu.async_copy(xdn, xd_hbm.at[es], sw),
                pltpu.async_copy(xsn, xs_hbm.at[es], sw),
            ]

        @pl.loop(0, npairs)
        def _(j):
            e0 = base + j * (2 * CG)
            e1 = e0 + CG
            ca0, cb0 = fetch(e0, CG, si0, di0, ab0, bb0, sa0, sb0)
            ca1, cb1 = fetch(e1, CG, si1, di1, ab1, bb1, sa1, sb1)
            ca0.wait()
            cb0.wait()
            w0 = wback(e0, CG, ab0, bb0, xd0, xs0, sw0)
            ca1.wait()
            cb1.wait()
            w1 = wback(e1, CG, ab1, bb1, xd1, xs1, sw1)
            for w in w0 + w1:
                w.wait()

        tb = base + npairs * 2 * CG
        if rem:
            ca, cb = fetch(tb, CG, si0, di0, ab0, bb0, sa0, sb0)
            ca.wait()
            cb.wait()
            for w in wback(tb, CG, ab0, bb0, xd0, xs0, sw0):
                w.wait()
            tb = tb + CG
        if ctail:
            ca, cb = fetch(tb, ctail, si1, di1, ab1, bb1, sa1, sb1)
            ca.wait()
            cb.wait()
            for w in wback(tb, ctail, ab1, bb1, xd1, xs1, sw1):
                w.wait()

    return k(td, ts, src, dst)


def _sc_scatter_m(m2, dst, N):
    E = dst.shape[0]
    ept = E // _NSUB
    CS = 64
    nfull = ept // CS
    ctail = ept - nfull * CS
    npairs = nfull // 2
    rem = nfull % 2
    rpt = (N // _NSUB) // 8 * 8
    tail = N - _NSUB * rpt
    mesh = plsc.VectorSubcoreMesh(core_axis_name="c", subcore_axis_name="s")
    z128 = jnp.zeros((N, 128), jnp.float32)

    @functools.partial(
        pl.kernel, mesh=mesh,
        out_type=jax.ShapeDtypeStruct((2, N, 128), jnp.float32),
        scratch_types=[pltpu.VMEM((CS,), jnp.int32),
                       pltpu.VMEM((CS,), jnp.int32),
                       pltpu.VMEM((max(ctail, 8),), jnp.int32),
                       pltpu.VMEM((CS, 128), jnp.float32),
                       pltpu.VMEM((CS, 128), jnp.float32),
                       pltpu.VMEM((max(ctail, 8), 128), jnp.float32),
                       pltpu.VMEM_SHARED((N, 128), jnp.float32),
                       pltpu.SemaphoreType.DMA, pltpu.SemaphoreType.DMA,
                       pltpu.SemaphoreType.DMA, pltpu.SemaphoreType.DMA])
    def k(m_hbm, dst_hbm, z_hbm, agg_hbm,
          di0, di1, dit, rows0, rows1, rowst, acc, sr0, sr1, sd0, sd1):
        cid = lax.axis_index("c")
        sid = lax.axis_index("s")

        @pl.when(sid == 0)
        def _():
            pltpu.sync_copy(z_hbm, acc)

        plsc.subcore_barrier()
        base = sid * ept

        def fetch(eb, n, di, rows, sr):
            pltpu.sync_copy(dst_hbm.at[pl.ds(eb, n)], di)
            cr = pltpu.async_copy(m_hbm.at[cid, pl.ds(eb, n)], rows, sr)
            return cr, di, rows

        @pl.loop(0, npairs)
        def _(j):
            e0 = base + j * (2 * CS)
            c0, d0, r0 = fetch(e0, CS, di0, rows0, sr0)
            c1, d1, r1 = fetch(e0 + CS, CS, di1, rows1, sr1)
            c0.wait()
            a0 = pltpu.async_copy(r0, acc.at[d0], sd0, add=True)
            c1.wait()
            a1 = pltpu.async_copy(r1, acc.at[d1], sd1, add=True)
            a0.wait()
            a1.wait()

        tb = base + npairs * 2 * CS
        if rem:
            c0, d0, r0 = fetch(tb, CS, di0, rows0, sr0)
            c0.wait()
            pltpu.async_copy(r0, acc.at[d0], sd0, add=True).wait()
            tb = tb + CS
        if ctail:
            c1, d1, r1 = fetch(tb, ctail, dit, rowst, sr1)
            c1.wait()
            pltpu.async_copy(r1, acc.at[d1], sd1, add=True).wait()

        plsc.subcore_barrier()
        rb = sid * rpt
        pltpu.sync_copy(acc.at[pl.ds(rb, rpt)],
                        agg_hbm.at[cid, pl.ds(rb, rpt)])
        if tail:
            @pl.when(sid == 0)
            def _():
                tn = _NSUB * rpt
                pltpu.sync_copy(acc.at[pl.ds(tn, tail)],
                                agg_hbm.at[cid, pl.ds(tn, tail)])

    return k(m2, dst, z128)


def _sc_scatter_rc(rc, dst, N):
    E = dst.shape[0]
    eph = E // 2
    ept = eph // _NSUB
    CS = 64
    nfull = ept // CS
    ctail = ept - nfull * CS
    npairs = nfull // 2
    rem = nfull % 2
    rpt = (N // _NSUB) // 8 * 8
    tail = N - _NSUB * rpt
    mesh = plsc.VectorSubcoreMesh(core_axis_name="c", subcore_axis_name="s")
    z128 = jnp.zeros((N, 128), jnp.float32)

    @functools.partial(
        pl.kernel, mesh=mesh,
        out_type=jax.ShapeDtypeStruct((2, N, 128), jnp.float32),
        scratch_types=[pltpu.VMEM((CS,), jnp.int32),
                       pltpu.VMEM((CS,), jnp.int32),
                       pltpu.VMEM((max(ctail, 8),), jnp.int32),
                       pltpu.VMEM((CS, 128), jnp.float32),
                       pltpu.VMEM((CS, 128), jnp.float32),
                       pltpu.VMEM((max(ctail, 8), 128), jnp.float32),
                       pltpu.VMEM_SHARED((N, 128), jnp.float32),
                       pltpu.SemaphoreType.DMA, pltpu.SemaphoreType.DMA,
                       pltpu.SemaphoreType.DMA, pltpu.SemaphoreType.DMA])
    def k(rc_hbm, dst_hbm, z_hbm, dx_hbm,
          di0, di1, dit, rows0, rows1, rowst, acc, sr0, sr1, sd0, sd1):
        cid = lax.axis_index("c")
        sid = lax.axis_index("s")

        @pl.when(sid == 0)
        def _():
            pltpu.sync_copy(z_hbm, acc)

        plsc.subcore_barrier()
        base = cid * eph + sid * ept

        def fetch(eb, n, di, rows, sr):
            pltpu.sync_copy(dst_hbm.at[pl.ds(eb, n)], di)
            cr = pltpu.async_copy(rc_hbm.at[pl.ds(eb, n)], rows, sr)
            return cr, di, rows

        @pl.loop(0, npairs)
        def _(j):
            e0 = base + j * (2 * CS)
            c0, d0, r0 = fetch(e0, CS, di0, rows0, sr0)
            c1, d1, r1 = fetch(e0 + CS, CS, di1, rows1, sr1)
            c0.wait()
            a0 = pltpu.async_copy(r0, acc.at[d0], sd0, add=True)
            c1.wait()
            a1 = pltpu.async_copy(r1, acc.at[d1], sd1, add=True)
            a0.wait()
            a1.wait()

        tb = base + npairs * 2 * CS
        if rem:
            c0, d0, r0 = fetch(tb, CS, di0, rows0, sr0)
            c0.wait()
            pltpu.async_copy(r0, acc.at[d0], sd0, add=True).wait()
            tb = tb + CS
        if ctail:
            c1, d1, r1 = fetch(tb, ctail, dit, rowst, sr1)
            c1.wait()
            pltpu.async_copy(r1, acc.at[d1], sd1, add=True).wait()

        plsc.subcore_barrier()
        rb = sid * rpt
        pltpu.sync_copy(acc.at[pl.ds(rb, rpt)],
                        dx_hbm.at[cid, pl.ds(rb, rpt)])
        if tail:
            @pl.when(sid == 0)
            def _():
                tn = _NSUB * rpt
                pltpu.sync_copy(acc.at[pl.ds(tn, tail)],
                                dx_hbm.at[cid, pl.ds(tn, tail)])

    return k(rc, dst, z128)


def _egnn_block(h, xpad, feats, src, dst, blk, wext, i, with_x):
    N = h.shape[0]
    whd = blk['We1'][i, :_D, :]
    whs = blk['We1'][i, _D:2 * _D, :]
    wx_row = blk['Wx'][i].reshape(1, _D)
    w1a = blk['Wh1'][i, :_D, :]
    w1ba = blk['Wh1'][i, _D:_D + 128, :]
    w1bb = blk['Wh1'][i, _D + 128:, :]

    td, ts = _ab_prep(h, xpad, whd, whs)
    ga, gb, xd, xs = _sc_gather(td, ts, src, dst)
    res = _edge_mlp(ga, gb, xd, xs, feats, wext[i], blk['We2'][i],
                    blk['be2'][i], wx_row, with_coef=with_x)
    m2 = res[0]
    ag2 = _sc_scatter_m(m2, dst, N)
    dx2 = _sc_scatter_rc(res[1], dst, N) if with_x else None
    h_new, x_new = _node_update(h, ag2, w1a, w1ba, w1bb, blk['bh1'][i],
                                blk['Wh2'][i], blk['bh2'][i], xpad, dx2,
                                with_x)
    return h_new, x_new


def kernel(xp, edge_index_p, ep_feats, coord_p, xl, edge_index_l, el_feats,
           coord_l, edge_index_c, ec_feats, coord_c, params):
    NP = xp.shape[0]
    L = params['blk_p']['We1'].shape[0]

    def pad_x(c):
        n = c.shape[0]
        return jnp.concatenate(
            [c, jnp.zeros((n, _XW - c.shape[1]), jnp.float32)], axis=1)

    sp, dp = (edge_index_p[0].astype(jnp.int32),
              edge_index_p[1].astype(jnp.int32))
    sl, dl = (edge_index_l[0].astype(jnp.int32),
              edge_index_l[1].astype(jnp.int32))
    sc, dc = (edge_index_c[0].astype(jnp.int32),
              edge_index_c[1].astype(jnp.int32))

    wext_p, wext_l, wext_c = _prep_weights(params)
    hp = _init_node(xp, params['Wp_node'], params['bp_node'],
                    params['ln_p_g'], params['ln_p_b'])
    hl = _init_node(xl, params['Wl_node'], params['bl_node'],
                    params['ln_l_g'], params['ln_l_b'])
    xpp, xpl, xpc = pad_x(coord_p), pad_x(coord_l), pad_x(coord_c)

    for i in range(L):
        last = i == L - 1
        hp, xpp = _egnn_block(hp, xpp, ep_feats, sp, dp, params['blk_p'],
                              wext_p, i, with_x=not last)
        hl, xpl = _egnn_block(hl, xpl, el_feats, sl, dl, params['blk_l'],
                              wext_l, i, with_x=not last)
        hc = jnp.concatenate([hp, hl], axis=0)
        hc, xpc = _egnn_block(hc, xpc, ec_feats, sc, dc, params['blk_c'],
                              wext_c, i, with_x=not last)
        hp = hc[:NP]
        hl = hc[NP:]

    return hp, hl, hc
